# Initial kernel scaffold; baseline (speedup 1.0000x reference)
import jax
import jax.numpy as jnp
from jax.experimental import pallas as pl


def kernel(x, Wq, bq, Wk, bk, pos_bias):
    raise NotImplementedError("write your pallas kernel here")


# speedup vs baseline: 15.9524x; 15.9524x over previous
import functools

import jax
import jax.numpy as jnp
from jax.experimental import pallas as pl

K = 8
TEMP = 0.1
NEG_MASK = -1.0e9
NEG_INF = -3.0e38


def _proj_kernel(x_ref, wq_ref, bq_ref, wk_ref, bk_ref, q_ref, k_ref):
    xb = x_ref[...]
    dn = (((1,), (1,)), ((), ()))
    q = jax.lax.dot_general(xb, wq_ref[...], dn,
                            preferred_element_type=jnp.float32)
    q = q + bq_ref[...]
    qn = jnp.sqrt(jnp.sum(q * q, axis=1, keepdims=True))
    q_ref[...] = q / jnp.maximum(qn, 1e-12)
    k = jax.lax.dot_general(xb, wk_ref[...], dn,
                            preferred_element_type=jnp.float32)
    k = k + bk_ref[...]
    kn = jnp.sqrt(jnp.sum(k * k, axis=1, keepdims=True))
    k_ref[...] = k / jnp.maximum(kn, 1e-12)


def _route_kernel(q_ref, k_ref, pb_ref, routes_ref, w_ref, *, rb, p):
    q = q_ref[0]
    k = k_ref[0]
    dn = (((1,), (1,)), ((), ()))
    s = jax.lax.dot_general(q, k, dn, preferred_element_type=jnp.float32)
    s = s + pb_ref[...]
    r0 = pl.program_id(1) * rb
    rows = r0 + jax.lax.broadcasted_iota(jnp.int32, (rb, p), 0)
    cols = jax.lax.broadcasted_iota(jnp.int32, (rb, p), 1)
    s = jnp.where(rows == cols, NEG_MASK, s)
    vals = []
    idxs = []
    for _ in range(K):
        m = jnp.max(s, axis=1, keepdims=True)
        idx = jnp.min(jnp.where(s == m, cols, p), axis=1, keepdims=True)
        vals.append(m)
        idxs.append(idx)
        s = jnp.where(cols == idx, NEG_INF, s)
    v = jnp.concatenate(vals, axis=1)
    i = jnp.concatenate(idxs, axis=1)
    e = jnp.exp((v - v[:, :1]) * (1.0 / TEMP))
    w_ref[0] = e / jnp.sum(e, axis=1, keepdims=True)
    routes_ref[0] = i.astype(jnp.int32)


@jax.jit
def kernel(x, Wq, bq, Wk, bk, pos_bias):
    b, s_len, d = x.shape
    p = s_len - 1
    xin = x[:, 1:, :].reshape(b * p, d)

    rb1 = min(512, p)
    qf, kf = pl.pallas_call(
        _proj_kernel,
        grid=(b * p // rb1,),
        in_specs=[
            pl.BlockSpec((rb1, d), lambda i: (i, 0)),
            pl.BlockSpec((d, d), lambda i: (0, 0)),
            pl.BlockSpec((1, d), lambda i: (0, 0)),
            pl.BlockSpec((d, d), lambda i: (0, 0)),
            pl.BlockSpec((1, d), lambda i: (0, 0)),
        ],
        out_specs=[
            pl.BlockSpec((rb1, d), lambda i: (i, 0)),
            pl.BlockSpec((rb1, d), lambda i: (i, 0)),
        ],
        out_shape=[
            jax.ShapeDtypeStruct((b * p, d), jnp.float32),
            jax.ShapeDtypeStruct((b * p, d), jnp.float32),
        ],
    )(xin, Wq, bq.reshape(1, d), Wk, bk.reshape(1, d))

    qf = qf.reshape(b, p, d)
    kf = kf.reshape(b, p, d)
    pb = pos_bias[:p, :p]

    rb2 = min(256, p)
    routes, weights = pl.pallas_call(
        functools.partial(_route_kernel, rb=rb2, p=p),
        grid=(b, p // rb2),
        in_specs=[
            pl.BlockSpec((1, rb2, d), lambda bi, ri: (bi, ri, 0)),
            pl.BlockSpec((1, p, d), lambda bi, ri: (bi, 0, 0)),
            pl.BlockSpec((rb2, p), lambda bi, ri: (ri, 0)),
        ],
        out_specs=[
            pl.BlockSpec((1, rb2, K), lambda bi, ri: (bi, ri, 0)),
            pl.BlockSpec((1, rb2, K), lambda bi, ri: (bi, ri, 0)),
        ],
        out_shape=[
            jax.ShapeDtypeStruct((b, p, K), jnp.int32),
            jax.ShapeDtypeStruct((b, p, K), jnp.float32),
        ],
    )(qf, kf, pb)
    return routes, weights


# speedup vs baseline: 18.3946x; 1.1531x over previous
import functools

import jax
import jax.numpy as jnp
from jax.experimental import pallas as pl

K = 8
TEMP = 0.1
NEG_MASK = -1.0e9
NEG_INF = -3.0e38


def _proj_kernel(x_ref, wq_ref, bq_ref, wk_ref, bk_ref, q_ref, k_ref):
    xb = x_ref[...]
    dn = (((1,), (1,)), ((), ()))
    q = jax.lax.dot_general(xb, wq_ref[...], dn,
                            preferred_element_type=jnp.float32)
    q = q + bq_ref[...]
    qn = jnp.sqrt(jnp.sum(q * q, axis=1, keepdims=True))
    q_ref[...] = q / jnp.maximum(qn, 1e-12)
    k = jax.lax.dot_general(xb, wk_ref[...], dn,
                            preferred_element_type=jnp.float32)
    k = k + bk_ref[...]
    kn = jnp.sqrt(jnp.sum(k * k, axis=1, keepdims=True))
    k_ref[...] = k / jnp.maximum(kn, 1e-12)


def _route_kernel(q_ref, k_ref, pb_ref, routes_ref, w_ref, *, rb, p):
    q = q_ref[0]
    k = k_ref[0]
    dn = (((1,), (1,)), ((), ()))
    s = jax.lax.dot_general(q, k, dn, preferred_element_type=jnp.float32)
    s = s + pb_ref[...]
    colsf = jax.lax.broadcasted_iota(jnp.int32, (rb, p), 1).astype(jnp.float32)
    pf = float(p)
    vals = []
    idxs = []
    for _ in range(K):
        m = jnp.max(s, axis=1, keepdims=True)
        idx = jnp.min(jnp.where(s == m, colsf, pf), axis=1, keepdims=True)
        vals.append(m)
        idxs.append(idx)
        s = jnp.where(colsf == idx, NEG_INF, s)
    v = jnp.concatenate(vals, axis=1)
    i = jnp.concatenate(idxs, axis=1)
    e = jnp.exp((v - v[:, :1]) * (1.0 / TEMP))
    w_ref[0] = e / jnp.sum(e, axis=1, keepdims=True)
    routes_ref[0] = i.astype(jnp.int32)


@jax.jit
def kernel(x, Wq, bq, Wk, bk, pos_bias):
    b, s_len, d = x.shape
    p = s_len - 1
    xin = x[:, 1:, :].reshape(b * p, d)

    rb1 = min(512, p)
    qf, kf = pl.pallas_call(
        _proj_kernel,
        grid=(b * p // rb1,),
        in_specs=[
            pl.BlockSpec((rb1, d), lambda i: (i, 0)),
            pl.BlockSpec((d, d), lambda i: (0, 0)),
            pl.BlockSpec((1, d), lambda i: (0, 0)),
            pl.BlockSpec((d, d), lambda i: (0, 0)),
            pl.BlockSpec((1, d), lambda i: (0, 0)),
        ],
        out_specs=[
            pl.BlockSpec((rb1, d), lambda i: (i, 0)),
            pl.BlockSpec((rb1, d), lambda i: (i, 0)),
        ],
        out_shape=[
            jax.ShapeDtypeStruct((b * p, d), jnp.float32),
            jax.ShapeDtypeStruct((b * p, d), jnp.float32),
        ],
    )(xin, Wq, bq.reshape(1, d), Wk, bk.reshape(1, d))

    qf = qf.reshape(b, p, d)
    kf = kf.reshape(b, p, d)
    pb = pos_bias[:p, :p]

    rb2 = min(256, p)
    routes, weights = pl.pallas_call(
        functools.partial(_route_kernel, rb=rb2, p=p),
        grid=(b, p // rb2),
        in_specs=[
            pl.BlockSpec((1, rb2, d), lambda bi, ri: (bi, ri, 0)),
            pl.BlockSpec((1, p, d), lambda bi, ri: (bi, 0, 0)),
            pl.BlockSpec((rb2, p), lambda bi, ri: (ri, 0)),
        ],
        out_specs=[
            pl.BlockSpec((1, rb2, K), lambda bi, ri: (bi, ri, 0)),
            pl.BlockSpec((1, rb2, K), lambda bi, ri: (bi, ri, 0)),
        ],
        out_shape=[
            jax.ShapeDtypeStruct((b, p, K), jnp.int32),
            jax.ShapeDtypeStruct((b, p, K), jnp.float32),
        ],
    )(qf, kf, pb)
    return routes, weights


# speedup vs baseline: 20.7030x; 1.1255x over previous
import functools

import jax
import jax.numpy as jnp
from jax.experimental import pallas as pl
from jax.experimental.pallas import tpu as pltpu

K = 8
TEMP = 0.1
NEG_INF = -3.0e38


def _shifted_x(x_ref, xhi_ref):
    return jnp.concatenate([x_ref[0, 1:, :], xhi_ref[0, :1, :]], axis=0)


def _project(xb, w_ref, b_ref):
    dn = (((1,), (1,)), ((), ()))
    v = jax.lax.dot_general(xb, w_ref[...], dn,
                            preferred_element_type=jnp.float32)
    v = v + b_ref[...]
    n = jnp.sqrt(jnp.sum(v * v, axis=1, keepdims=True))
    return v / jnp.maximum(n, 1e-12)


def _kproj_kernel(x_ref, xhi_ref, wk_ref, bk_ref, k_ref):
    k_ref[0] = _project(_shifted_x(x_ref, xhi_ref), wk_ref, bk_ref)


def _route_kernel(x_ref, xhi_ref, wq_hbm, bq_ref, k_hbm, pb_ref,
                  routes_ref, w_ref, k_scr, wq_scr, k_sem, wq_sem,
                  *, rb, p, sub):
    @pl.when((pl.program_id(0) == 0) & (pl.program_id(1) == 0))
    def _():
        cp = pltpu.make_async_copy(wq_hbm, wq_scr, wq_sem)
        cp.start()
        cp.wait()

    @pl.when(pl.program_id(1) == 0)
    def _():
        cp = pltpu.make_async_copy(k_hbm.at[pl.program_id(0)], k_scr, k_sem)
        cp.start()
        cp.wait()

    q = _project(_shifted_x(x_ref, xhi_ref), wq_scr, bq_ref)
    k = k_scr[...]
    dn = (((1,), (1,)), ((), ()))
    colsf = jax.lax.broadcasted_iota(jnp.int32, (sub, p), 1).astype(jnp.float32)
    pf = float(p)
    for h in range(rb // sub):
        lo = h * sub
        s = jax.lax.dot_general(q[lo:lo + sub], k, dn,
                                preferred_element_type=jnp.float32)
        s = s + pb_ref[lo:lo + sub]
        vals = []
        idxs = []
        m = jnp.max(s, axis=1, keepdims=True)
        for j in range(K):
            idx = jnp.min(jnp.where(s == m, colsf, pf), axis=1, keepdims=True)
            vals.append(m)
            idxs.append(idx)
            if j < K - 1:
                s = jnp.where(colsf == idx, NEG_INF, s)
                m = jnp.max(s, axis=1, keepdims=True)
        v = jnp.concatenate(vals, axis=1)
        i = jnp.concatenate(idxs, axis=1)
        e = jnp.exp((v - v[:, :1]) * (1.0 / TEMP))
        w_ref[0, lo:lo + sub] = e / jnp.sum(e, axis=1, keepdims=True)
        routes_ref[0, lo:lo + sub] = i.astype(jnp.int32)


@jax.jit
def kernel(x, Wq, bq, Wk, bk, pos_bias):
    b, s_len, d = x.shape
    p = s_len - 1

    rb = min(512, p)
    hi_map = lambda bi, i, _r=rb // 8: (bi, (i + 1) * _r, 0)
    kf = pl.pallas_call(
        _kproj_kernel,
        grid=(b, p // rb),
        in_specs=[
            pl.BlockSpec((1, rb, d), lambda bi, i: (bi, i, 0)),
            pl.BlockSpec((1, 8, d), hi_map),
            pl.BlockSpec((d, d), lambda bi, i: (0, 0)),
            pl.BlockSpec((1, d), lambda bi, i: (0, 0)),
        ],
        out_specs=pl.BlockSpec((1, rb, d), lambda bi, i: (bi, i, 0)),
        out_shape=jax.ShapeDtypeStruct((b, p, d), jnp.float32),
    )(x, x, Wk, bk.reshape(1, d))
    pb = pos_bias[:p, :p]

    routes, weights = pl.pallas_call(
        functools.partial(_route_kernel, rb=rb, p=p, sub=min(256, rb)),
        grid=(b, p // rb),
        in_specs=[
            pl.BlockSpec((1, rb, d), lambda bi, ri: (bi, ri, 0)),
            pl.BlockSpec((1, 8, d), hi_map),
            pl.BlockSpec(memory_space=pl.ANY),
            pl.BlockSpec((1, d), lambda bi, ri: (0, 0)),
            pl.BlockSpec(memory_space=pl.ANY),
            pl.BlockSpec((rb, p), lambda bi, ri: (ri, 0)),
        ],
        scratch_shapes=[
            pltpu.VMEM((p, d), jnp.float32),
            pltpu.VMEM((d, d), jnp.float32),
            pltpu.SemaphoreType.DMA,
            pltpu.SemaphoreType.DMA,
        ],
        out_specs=[
            pl.BlockSpec((1, rb, K), lambda bi, ri: (bi, ri, 0)),
            pl.BlockSpec((1, rb, K), lambda bi, ri: (bi, ri, 0)),
        ],
        out_shape=[
            jax.ShapeDtypeStruct((b, p, K), jnp.int32),
            jax.ShapeDtypeStruct((b, p, K), jnp.float32),
        ],
    )(x, x, Wq, bq.reshape(1, d), kf, pb)
    return routes, weights


# speedup vs baseline: 21.0370x; 1.0161x over previous
import functools

import jax
import jax.numpy as jnp
from jax.experimental import pallas as pl
from jax.experimental.pallas import tpu as pltpu

K = 8
TEMP = 0.1
NEG_INF = -3.0e38


def _proj_kernel(x_ref, xhi_ref, wq_ref, bq_ref, wk_ref, bk_ref, q_ref, k_ref):
    xb = jnp.concatenate([x_ref[0, 1:, :], xhi_ref[0, :1, :]], axis=0)
    dn = (((1,), (1,)), ((), ()))
    q = jax.lax.dot_general(xb, wq_ref[...], dn,
                            preferred_element_type=jnp.float32)
    q = q + bq_ref[...]
    qn = jnp.sqrt(jnp.sum(q * q, axis=1, keepdims=True))
    q_ref[0] = q / jnp.maximum(qn, 1e-12)
    k = jax.lax.dot_general(xb, wk_ref[...], dn,
                            preferred_element_type=jnp.float32)
    k = k + bk_ref[...]
    kn = jnp.sqrt(jnp.sum(k * k, axis=1, keepdims=True))
    k_ref[0] = k / jnp.maximum(kn, 1e-12)


def _route_kernel(q_ref, k_hbm, pb_ref, routes_ref, w_ref, k_scr, k_sem,
                  *, rb, p, sub):
    @pl.when(pl.program_id(1) == 0)
    def _():
        cp = pltpu.make_async_copy(k_hbm.at[pl.program_id(0)], k_scr, k_sem)
        cp.start()
        cp.wait()

    q = q_ref[0]
    k = k_scr[...]
    dn = (((1,), (1,)), ((), ()))
    colsf = jax.lax.broadcasted_iota(jnp.int32, (sub, p), 1).astype(jnp.float32)
    pf = float(p)
    for h in range(rb // sub):
        lo = h * sub
        s = jax.lax.dot_general(q[lo:lo + sub], k, dn,
                                preferred_element_type=jnp.float32)
        s = s + pb_ref[lo:lo + sub]
        vals = []
        idxs = []
        m = jnp.max(s, axis=1, keepdims=True)
        for j in range(K):
            idx = jnp.min(jnp.where(s == m, colsf, pf), axis=1, keepdims=True)
            vals.append(m)
            idxs.append(idx)
            if j < K - 1:
                s = jnp.where(colsf == idx, NEG_INF, s)
                m = jnp.max(s, axis=1, keepdims=True)
        v = jnp.concatenate(vals, axis=1)
        i = jnp.concatenate(idxs, axis=1)
        e = jnp.exp((v - v[:, :1]) * (1.0 / TEMP))
        w_ref[0, lo:lo + sub] = e / jnp.sum(e, axis=1, keepdims=True)
        routes_ref[0, lo:lo + sub] = i.astype(jnp.int32)


@jax.jit
def kernel(x, Wq, bq, Wk, bk, pos_bias):
    b, s_len, d = x.shape
    p = s_len - 1

    rb = min(512, p)
    hi_map = lambda bi, i, _r=rb // 8: (bi, (i + 1) * _r, 0)
    qf, kf = pl.pallas_call(
        _proj_kernel,
        grid=(b, p // rb),
        in_specs=[
            pl.BlockSpec((1, rb, d), lambda bi, i: (bi, i, 0)),
            pl.BlockSpec((1, 8, d), hi_map),
            pl.BlockSpec((d, d), lambda bi, i: (0, 0)),
            pl.BlockSpec((1, d), lambda bi, i: (0, 0)),
            pl.BlockSpec((d, d), lambda bi, i: (0, 0)),
            pl.BlockSpec((1, d), lambda bi, i: (0, 0)),
        ],
        out_specs=[
            pl.BlockSpec((1, rb, d), lambda bi, i: (bi, i, 0)),
            pl.BlockSpec((1, rb, d), lambda bi, i: (bi, i, 0)),
        ],
        out_shape=[
            jax.ShapeDtypeStruct((b, p, d), jnp.float32),
            jax.ShapeDtypeStruct((b, p, d), jnp.float32),
        ],
    )(x, x, Wq, bq.reshape(1, d), Wk, bk.reshape(1, d))

    routes, weights = pl.pallas_call(
        functools.partial(_route_kernel, rb=rb, p=p, sub=min(256, rb)),
        grid=(b, p // rb),
        in_specs=[
            pl.BlockSpec((1, rb, d), lambda bi, ri: (bi, ri, 0)),
            pl.BlockSpec(memory_space=pl.ANY),
            pl.BlockSpec((rb, p), lambda bi, ri: (ri, 0)),
        ],
        scratch_shapes=[
            pltpu.VMEM((p, d), jnp.float32),
            pltpu.SemaphoreType.DMA,
        ],
        out_specs=[
            pl.BlockSpec((1, rb, K), lambda bi, ri: (bi, ri, 0)),
            pl.BlockSpec((1, rb, K), lambda bi, ri: (bi, ri, 0)),
        ],
        out_shape=[
            jax.ShapeDtypeStruct((b, p, K), jnp.int32),
            jax.ShapeDtypeStruct((b, p, K), jnp.float32),
        ],
    )(qf, kf, pos_bias)
    return routes, weights
